# baseline (speedup 1.0000x reference)
import functools

import jax
import jax.numpy as jnp
from jax import lax
from jax.experimental import pallas as pl
from jax.experimental.pallas import tpu as pltpu
from jax.experimental.pallas import tpu_sc as plsc

N = 10000
E = 160000
D = 256
H = 256
G = 64
HALF = 128

NC = 2
NS = 16
CK = 128
NCHUNK = 80
EPT = NCHUNK * CK
EPAD = NS * EPT
ROWS_A = 624

_SPLAT_DNUMS = lax.GatherDimensionNumbers(
    offset_dims=(), collapsed_slice_dims=(0,), start_index_map=(0,))


def _lane_splat(v16, j):
    idx = jnp.full((16, 1), j, dtype=jnp.int32)
    return lax.gather(v16, idx, _SPLAT_DNUMS, (1,),
                      mode=lax.GatherScatterMode.PROMISE_IN_BOUNDS)


def _make_sc_agg():
    mesh = plsc.VectorSubcoreMesh(core_axis_name="c", subcore_axis_name="s")

    @functools.partial(
        pl.kernel,
        out_type=[
            jax.ShapeDtypeStruct((N, HALF), jnp.float32),
            jax.ShapeDtypeStruct((N, HALF), jnp.float32),
        ],
        mesh=mesh,
        scratch_types=[
            pltpu.VMEM((NCHUNK, CK), jnp.float32),
            pltpu.VMEM((2, CK), jnp.int32),
            pltpu.VMEM((2, CK), jnp.int32),
            pltpu.VMEM((2, CK), jnp.int32),
            pltpu.VMEM((2, CK), jnp.int32),
            pltpu.VMEM((CK, HALF), jnp.float32),
            pltpu.VMEM((CK, HALF), jnp.float32),
            pltpu.VMEM_SHARED((N, HALF), jnp.float32),
            pltpu.SemaphoreType.DMA,
            pltpu.SemaphoreType.DMA,
            pltpu.SemaphoreType.DMA,
            pltpu.SemaphoreType.DMA,
            pltpu.SemaphoreType.DMA,
            pltpu.SemaphoreType.DMA,
        ],
    )
    def sc_agg(zlo_hbm, zhi_hbm, edata_hbm, ew_hbm,
               alo_hbm, ahi_hbm,
               eww, idx0, idx1, idx2, idx3, rows0, rows1, acc,
               gsem0, gsem1, isem0, isem1, isem2, isem3):
        c = lax.axis_index("c")
        s = lax.axis_index("s")
        idxs = [idx0, idx1, idx2, idx3]
        isems = [isem0, isem1, isem2, isem3]
        rows = [rows0, rows1]
        gsems = [gsem0, gsem1]

        eoff = pl.multiple_of(s * NCHUNK, 16)
        pltpu.sync_copy(ew_hbm.at[pl.ds(eoff, NCHUNK)], eww)

        zeros16 = jnp.zeros((16,), jnp.float32)

        def zfill(r, carry):
            for kk in range(HALF // 16):
                rows0[r, pl.ds(kk * 16, 16)] = zeros16
            return carry

        lax.fori_loop(0, CK, zfill, 0)
        roff = pl.multiple_of(s * ROWS_A, 16)
        for p in range(4):
            off = pl.multiple_of(roff + p * CK, 16)
            pltpu.sync_copy(rows0, acc.at[pl.ds(off, CK)])
        off = pl.multiple_of(roff + 4 * CK, 16)
        pltpu.sync_copy(rows0.at[pl.ds(0, ROWS_A - 4 * CK)],
                        acc.at[pl.ds(off, ROWS_A - 4 * CK)])

        @pl.when(s == NS - 1)
        def _():
            pltpu.sync_copy(rows0.at[pl.ds(0, 16)],
                            acc.at[pl.ds(N - 16, 16)])

        plsc.subcore_barrier()

        def idx_copy(j, ib, sem):
            pltpu.async_copy(edata_hbm.at[eoff + j], ib, sem)

        def wait_idx(ib, sem):
            pltpu.make_async_copy(edata_hbm.at[0], ib, sem).wait()

        def start_gather(ib, buf, sem):
            @pl.when(c == 0)
            def _():
                pltpu.async_copy(zlo_hbm.at[ib.at[0]], buf, sem)

            @pl.when(c == 1)
            def _():
                pltpu.async_copy(zhi_hbm.at[ib.at[0]], buf, sem)

        def wait_gather(buf, sem):
            pltpu.make_async_copy(zlo_hbm.at[pl.ds(0, CK)], buf, sem).wait()

        def scale(j, buf):
            def group(j0, carry):
                ew16 = eww[j, pl.ds(j0 * 16, 16)]
                for j1 in range(16):
                    w = _lane_splat(ew16, j1)
                    for kk in range(HALF // 16):
                        sl = pl.ds(kk * 16, 16)
                        buf[j0 * 16 + j1, sl] = buf[j0 * 16 + j1, sl] * w
                return carry

            lax.fori_loop(0, CK // 16, group, 0)

        for k in range(4):
            idx_copy(k, idxs[k], isems[k])
        for k in range(2):
            wait_idx(idxs[k], isems[k])
            start_gather(idxs[k], rows[k], gsems[k])

        def pipe(i, carry):
            a = i * 4
            for k in range(4):
                r = k % 2
                wait_gather(rows[r], gsems[r])
                scale(a + k, rows[r])
                pltpu.sync_copy(rows[r], acc.at[idxs[k].at[1]], add=True)

                @pl.when(a + k + 4 < NCHUNK)
                def _():
                    idx_copy(a + k + 4, idxs[k], isems[k])

                @pl.when(a + k + 2 < NCHUNK)
                def _():
                    wait_idx(idxs[(k + 2) % 4], isems[(k + 2) % 4])
                    start_gather(idxs[(k + 2) % 4], rows[r], gsems[r])

            return carry

        lax.fori_loop(0, NCHUNK // 4, pipe, 0)
        plsc.subcore_barrier()

        last = NS - 1
        tail_off = ROWS_A * last

        @pl.when(jnp.logical_and(c == 0, s < last))
        def _():
            pltpu.sync_copy(acc.at[pl.ds(roff, ROWS_A)],
                            alo_hbm.at[pl.ds(roff, ROWS_A)])

        @pl.when(jnp.logical_and(c == 0, s == last))
        def _():
            pltpu.sync_copy(acc.at[pl.ds(tail_off, N - tail_off)],
                            alo_hbm.at[pl.ds(tail_off, N - tail_off)])

        @pl.when(jnp.logical_and(c == 1, s < last))
        def _():
            pltpu.sync_copy(acc.at[pl.ds(roff, ROWS_A)],
                            ahi_hbm.at[pl.ds(roff, ROWS_A)])

        @pl.when(jnp.logical_and(c == 1, s == last))
        def _():
            pltpu.sync_copy(acc.at[pl.ds(tail_off, N - tail_off)],
                            ahi_hbm.at[pl.ds(tail_off, N - tail_off)])

    return sc_agg


_sc_agg = _make_sc_agg()

BN = 1000


def _tc_body(z_ref, alo_ref, ahi_ref, batch_ref,
             W1_ref, b1_ref, W2_ref, b2_ref, eps_ref,
             zout_ref, zlo_ref, zhi_ref, g_ref):
    i = pl.program_id(0)
    eps = eps_ref[0, 0]
    agg = jnp.concatenate([alo_ref[...], ahi_ref[...]], axis=1)
    h = (1.0 + eps) * z_ref[...] + agg
    h = jnp.maximum(
        jnp.dot(h, W1_ref[...], preferred_element_type=jnp.float32)
        + b1_ref[...], 0.0)
    h = jnp.dot(h, W2_ref[...], preferred_element_type=jnp.float32) + b2_ref[...]
    zn = jnp.maximum(h, 0.0)
    zout_ref[...] = zn
    zlo_ref[...] = zn[:, :HALF]
    zhi_ref[...] = zn[:, HALF:]
    onehot = (lax.broadcasted_iota(jnp.int32, (G, BN), 0)
              == batch_ref[0]).astype(jnp.float32)
    part = jnp.dot(onehot, zn, preferred_element_type=jnp.float32)

    @pl.when(i == 0)
    def _():
        g_ref[...] = jnp.zeros_like(g_ref)

    g_ref[...] += part


_tc_mlp = pl.pallas_call(
    _tc_body,
    grid=(N // BN,),
    in_specs=[
        pl.BlockSpec((BN, D), lambda i: (i, 0)),
        pl.BlockSpec((BN, HALF), lambda i: (i, 0)),
        pl.BlockSpec((BN, HALF), lambda i: (i, 0)),
        pl.BlockSpec((1, 1, BN), lambda i: (i, 0, 0)),
        pl.BlockSpec((D, H), lambda i: (0, 0)),
        pl.BlockSpec((1, H), lambda i: (0, 0)),
        pl.BlockSpec((H, H), lambda i: (0, 0)),
        pl.BlockSpec((1, H), lambda i: (0, 0)),
        pl.BlockSpec((1, 1), lambda i: (0, 0)),
    ],
    out_specs=[
        pl.BlockSpec((BN, H), lambda i: (i, 0)),
        pl.BlockSpec((BN, HALF), lambda i: (i, 0)),
        pl.BlockSpec((BN, HALF), lambda i: (i, 0)),
        pl.BlockSpec((G, H), lambda i: (0, 0)),
    ],
    out_shape=[
        jax.ShapeDtypeStruct((N, H), jnp.float32),
        jax.ShapeDtypeStruct((N, HALF), jnp.float32),
        jax.ShapeDtypeStruct((N, HALF), jnp.float32),
        jax.ShapeDtypeStruct((G, H), jnp.float32),
    ],
)


def kernel(x, edge_index, edge_weights, batch,
           W1_0, b1_0, W2_0, b2_0, eps_0,
           W1_1, b1_1, W2_1, b2_1, eps_1,
           W1_2, b1_2, W2_2, b2_2, eps_2):
    params = [(W1_0, b1_0, W2_0, b2_0, eps_0),
              (W1_1, b1_1, W2_1, b2_1, eps_1),
              (W1_2, b1_2, W2_2, b2_2, eps_2)]
    pad = EPAD - E
    order = jnp.argsort(edge_index[0])
    edge_index = edge_index[:, order]
    edge_weights = edge_weights[order]
    src2 = jnp.concatenate([edge_index[0],
                            jnp.zeros((pad,), jnp.int32)]).reshape(-1, CK)
    dst2 = jnp.concatenate([edge_index[1],
                            jnp.zeros((pad,), jnp.int32)]).reshape(-1, CK)
    edata = jnp.stack([src2, dst2], axis=1)
    ew = jnp.concatenate([edge_weights,
                          jnp.zeros((pad,), jnp.float32)]).reshape(-1, CK)
    batch2d = batch.reshape(N // BN, 1, BN)
    z = x
    zlo = x[:, :HALF]
    zhi = x[:, HALF:]
    gs = []
    for (W1, b1, W2, b2, eps) in params:
        alo, ahi = _sc_agg(zlo, zhi, edata, ew)
        z, zlo, zhi, g = _tc_mlp(z, alo, ahi, batch2d,
                                 W1, b1.reshape(1, H), W2, b2.reshape(1, H),
                                 eps.reshape(1, 1))
        gs.append(g)
    return (z, jnp.concatenate(gs, axis=1))


# speedup vs baseline: 1.3992x; 1.3992x over previous
import functools

import jax
import jax.numpy as jnp
from jax import lax
from jax.experimental import pallas as pl
from jax.experimental.pallas import tpu as pltpu
from jax.experimental.pallas import tpu_sc as plsc

N = 10000
E = 160000
D = 256
H = 256
G = 64
HALF = 128

NC = 2
NS = 16
CK = 128
NCHUNK = 80
EPT = NCHUNK * CK
EPAD = NS * EPT
ROWS_A = 624

_SPLAT_DNUMS = lax.GatherDimensionNumbers(
    offset_dims=(), collapsed_slice_dims=(0,), start_index_map=(0,))


def _lane_splat(v16, j):
    idx = jnp.full((16, 1), j, dtype=jnp.int32)
    return lax.gather(v16, idx, _SPLAT_DNUMS, (1,),
                      mode=lax.GatherScatterMode.PROMISE_IN_BOUNDS)


def _make_sc_agg():
    mesh = plsc.VectorSubcoreMesh(core_axis_name="c", subcore_axis_name="s")

    @functools.partial(
        pl.kernel,
        out_type=[
            jax.ShapeDtypeStruct((N, HALF), jnp.float32),
            jax.ShapeDtypeStruct((N, HALF), jnp.float32),
        ],
        mesh=mesh,
        scratch_types=[
            pltpu.VMEM((NCHUNK, CK), jnp.float32),
            pltpu.VMEM((2, CK), jnp.int32),
            pltpu.VMEM((2, CK), jnp.int32),
            pltpu.VMEM((2, CK), jnp.int32),
            pltpu.VMEM((2, CK), jnp.int32),
            pltpu.VMEM((CK, HALF), jnp.float32),
            pltpu.VMEM((CK, HALF), jnp.float32),
            pltpu.VMEM_SHARED((N, HALF), jnp.float32),
            pltpu.SemaphoreType.DMA,
            pltpu.SemaphoreType.DMA,
            pltpu.SemaphoreType.DMA,
            pltpu.SemaphoreType.DMA,
            pltpu.SemaphoreType.DMA,
            pltpu.SemaphoreType.DMA,
        ],
    )
    def sc_agg(zlo_hbm, zhi_hbm, edata_hbm, ew_hbm,
               alo_hbm, ahi_hbm,
               eww, idx0, idx1, idx2, idx3, rows0, rows1, acc,
               gsem0, gsem1, isem0, isem1, isem2, isem3):
        c = lax.axis_index("c")
        s = lax.axis_index("s")
        idxs = [idx0, idx1, idx2, idx3]
        isems = [isem0, isem1, isem2, isem3]
        rows = [rows0, rows1]
        gsems = [gsem0, gsem1]

        eoff = pl.multiple_of(s * NCHUNK, 16)
        pltpu.sync_copy(ew_hbm.at[pl.ds(eoff, NCHUNK)], eww)

        zeros16 = jnp.zeros((16,), jnp.float32)

        def zfill(r, carry):
            for kk in range(HALF // 16):
                rows0[r, pl.ds(kk * 16, 16)] = zeros16
            return carry

        lax.fori_loop(0, CK, zfill, 0)
        roff = pl.multiple_of(s * ROWS_A, 16)
        for p in range(4):
            off = pl.multiple_of(roff + p * CK, 16)
            pltpu.sync_copy(rows0, acc.at[pl.ds(off, CK)])
        off = pl.multiple_of(roff + 4 * CK, 16)
        pltpu.sync_copy(rows0.at[pl.ds(0, ROWS_A - 4 * CK)],
                        acc.at[pl.ds(off, ROWS_A - 4 * CK)])

        @pl.when(s == NS - 1)
        def _():
            pltpu.sync_copy(rows0.at[pl.ds(0, 16)],
                            acc.at[pl.ds(N - 16, 16)])

        plsc.subcore_barrier()

        def idx_copy(j, ib, sem):
            pltpu.async_copy(edata_hbm.at[eoff + j], ib, sem)

        def wait_idx(ib, sem):
            pltpu.make_async_copy(edata_hbm.at[0], ib, sem).wait()

        def start_gather(ib, buf, sem):
            @pl.when(c == 0)
            def _():
                pltpu.async_copy(zlo_hbm.at[ib.at[0]], buf, sem)

            @pl.when(c == 1)
            def _():
                pltpu.async_copy(zhi_hbm.at[ib.at[0]], buf, sem)

        def wait_gather(buf, sem):
            pltpu.make_async_copy(zlo_hbm.at[pl.ds(0, CK)], buf, sem).wait()

        def scale(j, buf):
            def group(j0, carry):
                ew16 = eww[j, pl.ds(j0 * 16, 16)]
                for j1 in range(16):
                    w = _lane_splat(ew16, j1)
                    for kk in range(HALF // 16):
                        sl = pl.ds(kk * 16, 16)
                        buf[j0 * 16 + j1, sl] = buf[j0 * 16 + j1, sl] * w
                return carry

            lax.fori_loop(0, CK // 16, group, 0)

        for k in range(4):
            idx_copy(k, idxs[k], isems[k])
        for k in range(2):
            wait_idx(idxs[k], isems[k])
            start_gather(idxs[k], rows[k], gsems[k])

        def pipe(i, carry):
            a = i * 4
            for k in range(4):
                r = k % 2
                wait_gather(rows[r], gsems[r])
                scale(a + k, rows[r])
                pltpu.sync_copy(rows[r], acc.at[idxs[k].at[1]], add=True)

                @pl.when(a + k + 4 < NCHUNK)
                def _():
                    idx_copy(a + k + 4, idxs[k], isems[k])

                @pl.when(a + k + 2 < NCHUNK)
                def _():
                    wait_idx(idxs[(k + 2) % 4], isems[(k + 2) % 4])
                    start_gather(idxs[(k + 2) % 4], rows[r], gsems[r])

            return carry

        lax.fori_loop(0, NCHUNK // 4, pipe, 0)
        plsc.subcore_barrier()

        last = NS - 1
        tail_off = ROWS_A * last

        @pl.when(jnp.logical_and(c == 0, s < last))
        def _():
            pltpu.sync_copy(acc.at[pl.ds(roff, ROWS_A)],
                            alo_hbm.at[pl.ds(roff, ROWS_A)])

        @pl.when(jnp.logical_and(c == 0, s == last))
        def _():
            pltpu.sync_copy(acc.at[pl.ds(tail_off, N - tail_off)],
                            alo_hbm.at[pl.ds(tail_off, N - tail_off)])

        @pl.when(jnp.logical_and(c == 1, s < last))
        def _():
            pltpu.sync_copy(acc.at[pl.ds(roff, ROWS_A)],
                            ahi_hbm.at[pl.ds(roff, ROWS_A)])

        @pl.when(jnp.logical_and(c == 1, s == last))
        def _():
            pltpu.sync_copy(acc.at[pl.ds(tail_off, N - tail_off)],
                            ahi_hbm.at[pl.ds(tail_off, N - tail_off)])

    return sc_agg


_sc_agg = _make_sc_agg()

BN = 1000


def _tc_body(z_ref, alo_ref, ahi_ref, batch_ref,
             W1_ref, b1_ref, W2_ref, b2_ref, eps_ref,
             zout_ref, zlo_ref, zhi_ref, g_ref):
    i = pl.program_id(0)
    eps = eps_ref[0, 0]
    agg = jnp.concatenate([alo_ref[...], ahi_ref[...]], axis=1)
    h = (1.0 + eps) * z_ref[...] + agg
    h = jnp.maximum(
        jnp.dot(h, W1_ref[...], preferred_element_type=jnp.float32)
        + b1_ref[...], 0.0)
    h = jnp.dot(h, W2_ref[...], preferred_element_type=jnp.float32) + b2_ref[...]
    zn = jnp.maximum(h, 0.0)
    zout_ref[...] = zn
    zlo_ref[...] = zn[:, :HALF]
    zhi_ref[...] = zn[:, HALF:]
    onehot = (lax.broadcasted_iota(jnp.int32, (G, BN), 0)
              == batch_ref[0]).astype(jnp.float32)
    part = jnp.dot(onehot, zn, preferred_element_type=jnp.float32)

    @pl.when(i == 0)
    def _():
        g_ref[...] = jnp.zeros_like(g_ref)

    g_ref[...] += part


_tc_mlp = pl.pallas_call(
    _tc_body,
    grid=(N // BN,),
    in_specs=[
        pl.BlockSpec((BN, D), lambda i: (i, 0)),
        pl.BlockSpec((BN, HALF), lambda i: (i, 0)),
        pl.BlockSpec((BN, HALF), lambda i: (i, 0)),
        pl.BlockSpec((1, 1, BN), lambda i: (i, 0, 0)),
        pl.BlockSpec((D, H), lambda i: (0, 0)),
        pl.BlockSpec((1, H), lambda i: (0, 0)),
        pl.BlockSpec((H, H), lambda i: (0, 0)),
        pl.BlockSpec((1, H), lambda i: (0, 0)),
        pl.BlockSpec((1, 1), lambda i: (0, 0)),
    ],
    out_specs=[
        pl.BlockSpec((BN, H), lambda i: (i, 0)),
        pl.BlockSpec((BN, HALF), lambda i: (i, 0)),
        pl.BlockSpec((BN, HALF), lambda i: (i, 0)),
        pl.BlockSpec((G, H), lambda i: (0, 0)),
    ],
    out_shape=[
        jax.ShapeDtypeStruct((N, H), jnp.float32),
        jax.ShapeDtypeStruct((N, HALF), jnp.float32),
        jax.ShapeDtypeStruct((N, HALF), jnp.float32),
        jax.ShapeDtypeStruct((G, H), jnp.float32),
    ],
)


def kernel(x, edge_index, edge_weights, batch,
           W1_0, b1_0, W2_0, b2_0, eps_0,
           W1_1, b1_1, W2_1, b2_1, eps_1,
           W1_2, b1_2, W2_2, b2_2, eps_2):
    params = [(W1_0, b1_0, W2_0, b2_0, eps_0),
              (W1_1, b1_1, W2_1, b2_1, eps_1),
              (W1_2, b1_2, W2_2, b2_2, eps_2)]
    pad = EPAD - E
    src2 = jnp.concatenate([edge_index[0],
                            jnp.zeros((pad,), jnp.int32)]).reshape(-1, CK)
    dst2 = jnp.concatenate([edge_index[1],
                            jnp.zeros((pad,), jnp.int32)]).reshape(-1, CK)
    edata = jnp.stack([src2, dst2], axis=1)
    ew = jnp.concatenate([edge_weights,
                          jnp.zeros((pad,), jnp.float32)]).reshape(-1, CK)
    batch2d = batch.reshape(N // BN, 1, BN)
    z = x
    zlo = x[:, :HALF]
    zhi = x[:, HALF:]
    gs = []
    for (W1, b1, W2, b2, eps) in params:
        alo, ahi = _sc_agg(zlo, zhi, edata, ew)
        z, zlo, zhi, g = _tc_mlp(z, alo, ahi, batch2d,
                                 W1, b1.reshape(1, H), W2, b2.reshape(1, H),
                                 eps.reshape(1, 1))
        gs.append(g)
    return (z, jnp.concatenate(gs, axis=1))
